# baseline (speedup 1.0000x reference)
import functools

import jax
import jax.numpy as jnp
from jax import lax
from jax.experimental import pallas as pl
from jax.experimental.pallas import tpu as pltpu
from jax.experimental.pallas import tpu_sc as plsc

N = 5000
NPAD = 5120
RB = 256
RB2 = 128
CB = 640
D = 129
K = 32
H = 256
TILE = 224.0
THR = 0.5
BIGI = 2 ** 30

NC, NS = 2, 16
FG = 16
NHALF = 2
HALF = NPAD // NHALF
GCH = 512
SUBS = HALF // GCH
NCH = K * SUBS


def _topk_body(boxes_blk, boxesT, vals_out, idx_out, iou_s, vacc_s, iacc_s):
    rows = boxes_blk[...]
    bT = boxesT[...]
    x1c, y1c, x2c, y2c = bT[0:1, :], bT[1:2, :], bT[2:3, :], bT[3:4, :]
    x1r, y1r = rows[:, 0:1], rows[:, 1:2]
    x2r, y2r = rows[:, 2:3], rows[:, 3:4]
    inter = (jnp.maximum(jnp.minimum(x2r, x2c) - jnp.maximum(x1r, x1c), 0.0)
             * jnp.maximum(jnp.minimum(y2r, y2c) - jnp.maximum(y1r, y1c), 0.0))
    area_r = (x2r - x1r) * (y2r - y1r)
    area_c = (x2c - x1c) * (y2c - y1c)
    iou = inter / (area_r + area_c - inter + 1e-9)
    iou_s[...] = jnp.where(iou > THR, iou, -1.0)
    vacc_s[...] = jnp.full((RB, K), -1.0, dtype=jnp.float32)
    iacc_s[...] = jnp.zeros((RB, K), dtype=jnp.int32)

    colids = jax.lax.broadcasted_iota(jnp.int32, (RB, NPAD), 1)
    kio = jax.lax.broadcasted_iota(jnp.int32, (RB, K), 1)

    def body(carry):
        k, _ = carry
        cur = iou_s[...]
        m = jnp.max(cur, axis=1, keepdims=True)
        t = jnp.where(cur == m, colids, BIGI)
        a = jnp.min(t, axis=1, keepdims=True)
        vacc_s[...] = jnp.where(kio == k, m, vacc_s[...])
        iacc_s[...] = jnp.where(kio == k, a, iacc_s[...])
        iou_s[...] = jnp.where(colids == a, -1.0, cur)
        return k + 1, jnp.max(m) > 0.0

    def cond(carry):
        k, active = carry
        return jnp.logical_and(k < K, active)

    lax.while_loop(cond, body, (jnp.int32(0), True))
    vals_out[...] = vacc_s[...]
    idx_out[...] = iacc_s[...]


def _topk_pallas(boxes_p, boxesT):
    return pl.pallas_call(
        _topk_body,
        grid=(NPAD // RB,),
        in_specs=[
            pl.BlockSpec((RB, 4), lambda i: (i, 0)),
            pl.BlockSpec((8, NPAD), lambda i: (0, 0)),
        ],
        out_specs=[
            pl.BlockSpec((RB, K), lambda i: (i, 0)),
            pl.BlockSpec((RB, K), lambda i: (i, 0)),
        ],
        out_shape=[
            jax.ShapeDtypeStruct((NPAD, K), jnp.float32),
            jax.ShapeDtypeStruct((NPAD, K), jnp.int32),
        ],
        scratch_shapes=[
            pltpu.VMEM((RB, NPAD), jnp.float32),
            pltpu.VMEM((RB, K), jnp.float32),
            pltpu.VMEM((RB, K), jnp.int32),
        ],
    )(boxes_p, boxesT)


def _prep_body(featsT, boxesT, w1ct, w1nt, w1dst, b1c, t_out, base_out):
    f = featsT[...]
    bx = boxesT[...][0:4, :]
    p = jnp.dot(w1dst[...], bx, preferred_element_type=jnp.float32)
    t_out[...] = jnp.dot(w1nt[...], f, preferred_element_type=jnp.float32) + p
    base_out[...] = (jnp.dot(w1ct[...], f, preferred_element_type=jnp.float32)
                     + b1c[...] - p)


def _prep_pallas(featsT, boxesT, w1ct, w1nt, w1dst, b1c):
    return pl.pallas_call(
        _prep_body,
        grid=(NPAD // CB,),
        in_specs=[
            pl.BlockSpec((D, CB), lambda i: (0, i)),
            pl.BlockSpec((8, CB), lambda i: (0, i)),
            pl.BlockSpec((H, D), lambda i: (0, 0)),
            pl.BlockSpec((H, D), lambda i: (0, 0)),
            pl.BlockSpec((H, 4), lambda i: (0, 0)),
            pl.BlockSpec((H, 1), lambda i: (0, 0)),
        ],
        out_specs=[
            pl.BlockSpec((H, CB), lambda i: (0, i)),
            pl.BlockSpec((H, CB), lambda i: (0, i)),
        ],
        out_shape=[
            jax.ShapeDtypeStruct((H, NPAD), jnp.float32),
            jax.ShapeDtypeStruct((H, NPAD), jnp.float32),
        ],
    )(featsT, boxesT, w1ct, w1nt, w1dst, b1c)


def _make_sc_gather():
    mesh = plsc.VectorSubcoreMesh(core_axis_name="c", subcore_axis_name="s")

    @functools.partial(
        pl.kernel,
        mesh=mesh,
        out_type=jax.ShapeDtypeStruct((K, H, NPAD), jnp.float32),
        compiler_params=pltpu.CompilerParams(needs_layout_passes=False),
        scratch_types=[
            pltpu.VMEM((FG, NPAD), jnp.float32),
            pltpu.VMEM((GCH,), jnp.int32),
            pltpu.VMEM((GCH,), jnp.int32),
            pltpu.VMEM((FG, GCH), jnp.float32),
            pltpu.VMEM((FG, GCH), jnp.float32),
            pltpu.SemaphoreType.DMA,
            pltpu.SemaphoreType.DMA,
            pltpu.SemaphoreType.DMA,
            pltpu.SemaphoreType.DMA,
        ],
    )
    def sc_gather(tt_hbm, idx_hbm, out_hbm, tslice, idx0, idx1, obuf0, obuf1,
                  is0, is1, os0, os1):
        tid = lax.axis_index("s") * NC + lax.axis_index("c")
        g = tid // NHALF
        h = tid - g * NHALF
        pltpu.sync_copy(tt_hbm.at[pl.ds(g * FG, FG)], tslice)

        idxs = (idx0, idx1)
        obufs = (obuf0, obuf1)
        isems = (is0, is1)
        osems = (os0, os1)

        def gather_chunk(idx_v, obuf):
            for j in range(GCH // 16):
                idxv = idx_v[pl.ds(j * 16, 16)]
                for c in range(FG):
                    vals16 = plsc.load_gather(
                        tslice, [jnp.full((16,), c, jnp.int32), idxv])
                    obuf[c, pl.ds(j * 16, 16)] = vals16

        def out_slice(ch):
            k = ch // SUBS
            sub = ch - k * SUBS
            return out_hbm.at[k, pl.ds(g * FG, FG),
                              pl.ds(h * HALF + sub * GCH, GCH)]

        pltpu.async_copy(idx_hbm.at[h, 0], idx0, is0)

        def pair(i, carry):
            wbs = []
            for u in range(2):
                ch = 2 * i + u
                pltpu.make_async_copy(idx_hbm.at[h, ch], idxs[u],
                                      isems[u]).wait()
                nxt = ch + 1

                @pl.when(nxt < NCH)
                def _():
                    pltpu.async_copy(idx_hbm.at[h, nxt], idxs[1 - u],
                                     isems[1 - u])

                gather_chunk(idxs[u], obufs[u])
                wbs.append(pltpu.async_copy(obufs[u], out_slice(ch), osems[u]))
            for wb in wbs:
                wb.wait()
            return carry

        lax.fori_loop(0, NCH // 2, pair, 0)

    return sc_gather


@functools.cache
def _get_sc_gather():
    return _make_sc_gather()


def _sc_gather(tt, idx_sc):
    return _get_sc_gather()(tt, idx_sc)


def _make_mlp(final):
    def body(nt3, valsT, baseT, featsT, w1vc, w2t, b2c, wot, boc, wfc, bfc,
             out_ref):
        basev = baseT[...]
        w1vv = w1vc[...]
        w2v = w2t[...].astype(jnp.bfloat16)
        b2v = b2c[...]
        pooled = None
        for k in range(K):
            ntk = nt3[k]
            vk = valsT[k:k + 1, :]
            h1 = jnp.maximum(ntk + basev + w1vv * vk, 0.0)
            h2 = jnp.maximum(
                jnp.dot(w2v, h1.astype(jnp.bfloat16),
                        preferred_element_type=jnp.float32) + b2v,
                0.0)
            h2 = jnp.where(vk > THR, h2, -1e30)
            pooled = h2 if k == 0 else jnp.maximum(pooled, h2)
        pooled = jnp.maximum(pooled, 0.0)
        newf = (featsT[...]
                + jnp.dot(wot[...], pooled, preferred_element_type=jnp.float32)
                + boc[...])
        if final:
            s = jnp.sum(newf * wfc[...], axis=0, keepdims=True) + bfc[...]
            out_ref[...] = jnp.broadcast_to(s, (8, RB2))
        else:
            out_ref[...] = newf
    return body


def _mlp_pallas(nt3, valsT, baseT, featsT, w1vc, w2t, b2c, wot, boc, wfc, bfc,
                final):
    out_rows = 8 if final else D
    return pl.pallas_call(
        _make_mlp(final),
        grid=(NPAD // RB2,),
        in_specs=[
            pl.BlockSpec((K, H, RB2), lambda i: (0, 0, i)),
            pl.BlockSpec((K, RB2), lambda i: (0, i)),
            pl.BlockSpec((H, RB2), lambda i: (0, i)),
            pl.BlockSpec((D, RB2), lambda i: (0, i)),
            pl.BlockSpec((H, 1), lambda i: (0, 0)),
            pl.BlockSpec((H, H), lambda i: (0, 0)),
            pl.BlockSpec((H, 1), lambda i: (0, 0)),
            pl.BlockSpec((D, H), lambda i: (0, 0)),
            pl.BlockSpec((D, 1), lambda i: (0, 0)),
            pl.BlockSpec((D, 1), lambda i: (0, 0)),
            pl.BlockSpec((1, 1), lambda i: (0, 0)),
        ],
        out_specs=pl.BlockSpec((out_rows, RB2), lambda i: (0, i)),
        out_shape=jax.ShapeDtypeStruct((out_rows, NPAD), jnp.float32),
    )(nt3, valsT, baseT, featsT, w1vc, w2t, b2c, wot, boc, wfc, bfc)


def kernel(interpolated, rpn_boxes,
           W1_0, b1_0, W2_0, b2_0, Wo_0, bo_0,
           W1_1, b1_1, W2_1, b2_1, Wo_1, bo_1,
           Wf, bf):
    boxes_p = jnp.zeros((NPAD, 4), jnp.float32).at[:N].set(rpn_boxes)
    boxesT = jnp.zeros((8, NPAD), jnp.float32).at[:4].set(boxes_p.T)
    vals, idx = _topk_pallas(boxes_p, boxesT)
    valsT = vals.T
    idx_sc = (idx.T.reshape(K, NHALF, SUBS, GCH)
              .transpose(1, 0, 2, 3).reshape(NHALF, NCH, GCH))

    featsT = jnp.zeros((D, NPAD), jnp.float32).at[:, :N].set(interpolated.T)
    bfc = bf.reshape(1, 1)
    params = [(W1_0, b1_0, W2_0, b2_0, Wo_0, bo_0),
              (W1_1, b1_1, W2_1, b2_1, Wo_1, bo_1)]
    out = None
    for b, (W1, b1, W2, b2, Wo, bo) in enumerate(params):
        w1ct = W1[:D].T
        w1nt = W1[D:2 * D].T
        w1dst = (W1[2 * D:2 * D + 4] / TILE).T
        w1vc = W1[2 * D + 4].reshape(H, 1)
        tt, baseT = _prep_pallas(featsT, boxesT, w1ct, w1nt, w1dst,
                                 b1.reshape(H, 1))
        nt3 = _sc_gather(tt, idx_sc)
        final = b == len(params) - 1
        res = _mlp_pallas(nt3, valsT, baseT, featsT, w1vc, W2.T,
                          b2.reshape(H, 1), Wo.T, bo.reshape(D, 1), Wf, bfc,
                          final)
        if final:
            out = res
        else:
            featsT = res
    return out[0:1, :N].T


# speedup vs baseline: 1.0063x; 1.0063x over previous
import functools

import jax
import jax.numpy as jnp
from jax import lax
from jax.experimental import pallas as pl
from jax.experimental.pallas import tpu as pltpu
from jax.experimental.pallas import tpu_sc as plsc

N = 5000
NPAD = 5120
RB = 256
RB2 = 128
CB = 640
D = 129
K = 32
H = 256
TILE = 224.0
THR = 0.5
BIGI = 2 ** 30

NC, NS = 2, 16
FG = 16
NHALF = 2
HALF = NPAD // NHALF
GCH = 512
SUBS = HALF // GCH
NCH = K * SUBS
NCHP = NCH + 16


def _topk_body(boxes_blk, boxesT, vals_out, idx_out, iou_s, vacc_s, iacc_s):
    rows = boxes_blk[...]
    bT = boxesT[...]
    x1c, y1c, x2c, y2c = bT[0:1, :], bT[1:2, :], bT[2:3, :], bT[3:4, :]
    x1r, y1r = rows[:, 0:1], rows[:, 1:2]
    x2r, y2r = rows[:, 2:3], rows[:, 3:4]
    inter = (jnp.maximum(jnp.minimum(x2r, x2c) - jnp.maximum(x1r, x1c), 0.0)
             * jnp.maximum(jnp.minimum(y2r, y2c) - jnp.maximum(y1r, y1c), 0.0))
    area_r = (x2r - x1r) * (y2r - y1r)
    area_c = (x2c - x1c) * (y2c - y1c)
    iou = inter / (area_r + area_c - inter + 1e-9)
    iou_s[...] = jnp.where(iou > THR, iou, -1.0)
    vacc_s[...] = jnp.full((RB, K), -1.0, dtype=jnp.float32)
    iacc_s[...] = jnp.zeros((RB, K), dtype=jnp.int32)

    colids = jax.lax.broadcasted_iota(jnp.int32, (RB, NPAD), 1)
    kio = jax.lax.broadcasted_iota(jnp.int32, (RB, K), 1)

    def body(carry):
        k, _ = carry
        cur = iou_s[...]
        m = jnp.max(cur, axis=1, keepdims=True)
        t = jnp.where(cur == m, colids, BIGI)
        a = jnp.min(t, axis=1, keepdims=True)
        vacc_s[...] = jnp.where(kio == k, m, vacc_s[...])
        iacc_s[...] = jnp.where(kio == k, a, iacc_s[...])
        iou_s[...] = jnp.where(colids == a, -1.0, cur)
        return k + 1, jnp.max(m) > 0.0

    def cond(carry):
        k, active = carry
        return jnp.logical_and(k < K, active)

    lax.while_loop(cond, body, (jnp.int32(0), True))
    vals_out[...] = vacc_s[...]
    idx_out[...] = iacc_s[...]


def _topk_pallas(boxes_p, boxesT):
    return pl.pallas_call(
        _topk_body,
        grid=(NPAD // RB,),
        in_specs=[
            pl.BlockSpec((RB, 4), lambda i: (i, 0)),
            pl.BlockSpec((8, NPAD), lambda i: (0, 0)),
        ],
        out_specs=[
            pl.BlockSpec((RB, K), lambda i: (i, 0)),
            pl.BlockSpec((RB, K), lambda i: (i, 0)),
        ],
        out_shape=[
            jax.ShapeDtypeStruct((NPAD, K), jnp.float32),
            jax.ShapeDtypeStruct((NPAD, K), jnp.int32),
        ],
        scratch_shapes=[
            pltpu.VMEM((RB, NPAD), jnp.float32),
            pltpu.VMEM((RB, K), jnp.float32),
            pltpu.VMEM((RB, K), jnp.int32),
        ],
    )(boxes_p, boxesT)


def _prep_body(featsT, boxesT, w1ct, w1nt, w1dst, b1c, t_out, base_out):
    f = featsT[...]
    bx = boxesT[...][0:4, :]
    p = jnp.dot(w1dst[...], bx, preferred_element_type=jnp.float32)
    t_out[...] = jnp.dot(w1nt[...], f, preferred_element_type=jnp.float32) + p
    base_out[...] = (jnp.dot(w1ct[...], f, preferred_element_type=jnp.float32)
                     + b1c[...] - p)


def _prep_pallas(featsT, boxesT, w1ct, w1nt, w1dst, b1c):
    return pl.pallas_call(
        _prep_body,
        grid=(NPAD // CB,),
        in_specs=[
            pl.BlockSpec((D, CB), lambda i: (0, i)),
            pl.BlockSpec((8, CB), lambda i: (0, i)),
            pl.BlockSpec((H, D), lambda i: (0, 0)),
            pl.BlockSpec((H, D), lambda i: (0, 0)),
            pl.BlockSpec((H, 4), lambda i: (0, 0)),
            pl.BlockSpec((H, 1), lambda i: (0, 0)),
        ],
        out_specs=[
            pl.BlockSpec((H, CB), lambda i: (0, i)),
            pl.BlockSpec((H, CB), lambda i: (0, i)),
        ],
        out_shape=[
            jax.ShapeDtypeStruct((H, NPAD), jnp.float32),
            jax.ShapeDtypeStruct((H, NPAD), jnp.float32),
        ],
    )(featsT, boxesT, w1ct, w1nt, w1dst, b1c)


def _make_sc_gather():
    mesh = plsc.VectorSubcoreMesh(core_axis_name="c", subcore_axis_name="s")

    @functools.partial(
        pl.kernel,
        mesh=mesh,
        out_type=jax.ShapeDtypeStruct((K, H, NPAD), jnp.float32),
        compiler_params=pltpu.CompilerParams(needs_layout_passes=False),
        scratch_types=[
            pltpu.VMEM((FG, NPAD), jnp.float32),
            pltpu.VMEM((GCH,), jnp.int32),
            pltpu.VMEM((GCH,), jnp.int32),
            pltpu.VMEM((FG, GCH), jnp.float32),
            pltpu.VMEM((FG, GCH), jnp.float32),
            pltpu.VMEM((NCHP,), jnp.int32),
            pltpu.SemaphoreType.DMA,
            pltpu.SemaphoreType.DMA,
            pltpu.SemaphoreType.DMA,
            pltpu.SemaphoreType.DMA,
        ],
    )
    def sc_gather(tt_hbm, idx_hbm, flags_hbm, out_hbm, tslice, idx0, idx1,
                  obuf0, obuf1, flags_v, is0, is1, os0, os1):
        tid = lax.axis_index("s") * NC + lax.axis_index("c")
        g = tid // NHALF
        h = tid - g * NHALF
        pltpu.sync_copy(flags_hbm.at[h], flags_v)
        pltpu.sync_copy(tt_hbm.at[pl.ds(g * FG, FG)], tslice)

        def flag_at(ch):
            return flags_v[pl.ds(ch, 16)][0] > 0

        idxs = (idx0, idx1)
        obufs = (obuf0, obuf1)
        isems = (is0, is1)
        osems = (os0, os1)

        def gather_chunk(idx_v, obuf):
            for j in range(GCH // 16):
                idxv = idx_v[pl.ds(j * 16, 16)]
                for c in range(FG):
                    vals16 = plsc.load_gather(
                        tslice, [jnp.full((16,), c, jnp.int32), idxv])
                    obuf[c, pl.ds(j * 16, 16)] = vals16

        def out_slice(ch):
            k = ch // SUBS
            sub = ch - k * SUBS
            return out_hbm.at[k, pl.ds(g * FG, FG),
                              pl.ds(h * HALF + sub * GCH, GCH)]

        @pl.when(flag_at(0))
        def _():
            pltpu.async_copy(idx_hbm.at[h, 0], idx0, is0)

        def pair(i, carry):
            fs = []
            for u in range(2):
                ch = 2 * i + u
                f_ch = flag_at(ch)
                fs.append((u, ch, f_ch))

                @pl.when(f_ch)
                def _():
                    pltpu.make_async_copy(idx_hbm.at[h, ch], idxs[u],
                                          isems[u]).wait()

                nxt = ch + 1

                @pl.when(jnp.logical_and(nxt < NCH, flag_at(nxt)))
                def _():
                    pltpu.async_copy(idx_hbm.at[h, nxt], idxs[1 - u],
                                     isems[1 - u])

                @pl.when(f_ch)
                def _():
                    gather_chunk(idxs[u], obufs[u])
                    pltpu.async_copy(obufs[u], out_slice(ch), osems[u])

            for u, ch, f_ch in fs:

                @pl.when(f_ch)
                def _():
                    pltpu.make_async_copy(obufs[u], out_slice(ch),
                                          osems[u]).wait()
            return carry

        lax.fori_loop(0, NCH // 2, pair, 0)

    return sc_gather


@functools.cache
def _get_sc_gather():
    return _make_sc_gather()


def _sc_gather(tt, idx_sc, flags):
    return _get_sc_gather()(tt, idx_sc, flags)


def _make_mlp(final):
    def body(nt3, valsT, baseT, featsT, w1vc, w2t, b2c, wot, boc, wfc, bfc,
             out_ref):
        basev = baseT[...]
        w1vv = w1vc[...]
        w2v = w2t[...]
        b2v = b2c[...]
        pooled = None
        for k in range(K):
            ntk = nt3[k]
            vk = valsT[k:k + 1, :]
            h1 = jnp.maximum(ntk + basev + w1vv * vk, 0.0)
            h2 = jnp.maximum(
                jnp.dot(w2v, h1, preferred_element_type=jnp.float32) + b2v,
                0.0)
            h2 = jnp.where(vk > THR, h2, -1e30)
            pooled = h2 if k == 0 else jnp.maximum(pooled, h2)
        pooled = jnp.maximum(pooled, 0.0)
        newf = (featsT[...]
                + jnp.dot(wot[...], pooled, preferred_element_type=jnp.float32)
                + boc[...])
        if final:
            s = jnp.sum(newf * wfc[...], axis=0, keepdims=True) + bfc[...]
            out_ref[...] = jnp.broadcast_to(s, (8, RB2))
        else:
            out_ref[...] = newf
    return body


def _mlp_pallas(nt3, valsT, baseT, featsT, w1vc, w2t, b2c, wot, boc, wfc, bfc,
                final):
    out_rows = 8 if final else D
    return pl.pallas_call(
        _make_mlp(final),
        grid=(NPAD // RB2,),
        in_specs=[
            pl.BlockSpec((K, H, RB2), lambda i: (0, 0, i)),
            pl.BlockSpec((K, RB2), lambda i: (0, i)),
            pl.BlockSpec((H, RB2), lambda i: (0, i)),
            pl.BlockSpec((D, RB2), lambda i: (0, i)),
            pl.BlockSpec((H, 1), lambda i: (0, 0)),
            pl.BlockSpec((H, H), lambda i: (0, 0)),
            pl.BlockSpec((H, 1), lambda i: (0, 0)),
            pl.BlockSpec((D, H), lambda i: (0, 0)),
            pl.BlockSpec((D, 1), lambda i: (0, 0)),
            pl.BlockSpec((D, 1), lambda i: (0, 0)),
            pl.BlockSpec((1, 1), lambda i: (0, 0)),
        ],
        out_specs=pl.BlockSpec((out_rows, RB2), lambda i: (0, i)),
        out_shape=jax.ShapeDtypeStruct((out_rows, NPAD), jnp.float32),
    )(nt3, valsT, baseT, featsT, w1vc, w2t, b2c, wot, boc, wfc, bfc)


def kernel(interpolated, rpn_boxes,
           W1_0, b1_0, W2_0, b2_0, Wo_0, bo_0,
           W1_1, b1_1, W2_1, b2_1, Wo_1, bo_1,
           Wf, bf):
    boxes_p = jnp.zeros((NPAD, 4), jnp.float32).at[:N].set(rpn_boxes)
    boxesT = jnp.zeros((8, NPAD), jnp.float32).at[:4].set(boxes_p.T)
    vals, idx = _topk_pallas(boxes_p, boxesT)
    valsT = vals.T
    idx_sc = (idx.T.reshape(K, NHALF, SUBS, GCH)
              .transpose(1, 0, 2, 3).reshape(NHALF, NCH, GCH))
    act = ((valsT > THR).reshape(K, NHALF, SUBS, GCH).any(axis=-1)
           .transpose(1, 0, 2).reshape(NHALF, NCH).astype(jnp.int32))
    flags = jnp.zeros((NHALF, NCHP), jnp.int32).at[:, :NCH].set(act)

    featsT = jnp.zeros((D, NPAD), jnp.float32).at[:, :N].set(interpolated.T)
    bfc = bf.reshape(1, 1)
    params = [(W1_0, b1_0, W2_0, b2_0, Wo_0, bo_0),
              (W1_1, b1_1, W2_1, b2_1, Wo_1, bo_1)]
    out = None
    for b, (W1, b1, W2, b2, Wo, bo) in enumerate(params):
        w1ct = W1[:D].T
        w1nt = W1[D:2 * D].T
        w1dst = (W1[2 * D:2 * D + 4] / TILE).T
        w1vc = W1[2 * D + 4].reshape(H, 1)
        tt, baseT = _prep_pallas(featsT, boxesT, w1ct, w1nt, w1dst,
                                 b1.reshape(H, 1))
        nt3 = _sc_gather(tt, idx_sc, flags)
        final = b == len(params) - 1
        res = _mlp_pallas(nt3, valsT, baseT, featsT, w1vc, W2.T,
                          b2.reshape(H, 1), Wo.T, bo.reshape(D, 1), Wf, bfc,
                          final)
        if final:
            out = res
        else:
            featsT = res
    return out[0:1, :N].T


# speedup vs baseline: 1.0415x; 1.0349x over previous
import functools

import jax
import jax.numpy as jnp
from jax import lax
from jax.experimental import pallas as pl
from jax.experimental.pallas import tpu as pltpu
from jax.experimental.pallas import tpu_sc as plsc

N = 5000
NPAD = 5120
RB = 256
RB2 = 128
CB = 640
D = 129
K = 32
H = 256
TILE = 224.0
THR = 0.5
BIGI = 2 ** 30

NC, NS = 2, 16
FG = 16
NHALF = 2
HALF = NPAD // NHALF
GCH = 256
SUBS = HALF // GCH
NCH = K * SUBS
NCHP = NCH + 16


def _topk_body(boxes_blk, boxesT, vals_out, idx_out, iou_s, vacc_s, iacc_s):
    rows = boxes_blk[...]
    bT = boxesT[...]
    x1c, y1c, x2c, y2c = bT[0:1, :], bT[1:2, :], bT[2:3, :], bT[3:4, :]
    x1r, y1r = rows[:, 0:1], rows[:, 1:2]
    x2r, y2r = rows[:, 2:3], rows[:, 3:4]
    inter = (jnp.maximum(jnp.minimum(x2r, x2c) - jnp.maximum(x1r, x1c), 0.0)
             * jnp.maximum(jnp.minimum(y2r, y2c) - jnp.maximum(y1r, y1c), 0.0))
    area_r = (x2r - x1r) * (y2r - y1r)
    area_c = (x2c - x1c) * (y2c - y1c)
    iou = inter / (area_r + area_c - inter + 1e-9)
    iou_s[...] = jnp.where(iou > THR, iou, -1.0)
    vacc_s[...] = jnp.full((RB, K), -1.0, dtype=jnp.float32)
    iacc_s[...] = jnp.zeros((RB, K), dtype=jnp.int32)

    colids = jax.lax.broadcasted_iota(jnp.int32, (RB, NPAD), 1)
    kio = jax.lax.broadcasted_iota(jnp.int32, (RB, K), 1)

    def body(carry):
        k, _ = carry
        cur = iou_s[...]
        m = jnp.max(cur, axis=1, keepdims=True)
        t = jnp.where(cur == m, colids, BIGI)
        a = jnp.min(t, axis=1, keepdims=True)
        vacc_s[...] = jnp.where(kio == k, m, vacc_s[...])
        iacc_s[...] = jnp.where(kio == k, a, iacc_s[...])
        iou_s[...] = jnp.where(colids == a, -1.0, cur)
        return k + 1, jnp.max(m) > 0.0

    def cond(carry):
        k, active = carry
        return jnp.logical_and(k < K, active)

    lax.while_loop(cond, body, (jnp.int32(0), True))
    vals_out[...] = vacc_s[...]
    idx_out[...] = iacc_s[...]


def _topk_pallas(boxes_p, boxesT):
    return pl.pallas_call(
        _topk_body,
        grid=(NPAD // RB,),
        in_specs=[
            pl.BlockSpec((RB, 4), lambda i: (i, 0)),
            pl.BlockSpec((8, NPAD), lambda i: (0, 0)),
        ],
        out_specs=[
            pl.BlockSpec((RB, K), lambda i: (i, 0)),
            pl.BlockSpec((RB, K), lambda i: (i, 0)),
        ],
        out_shape=[
            jax.ShapeDtypeStruct((NPAD, K), jnp.float32),
            jax.ShapeDtypeStruct((NPAD, K), jnp.int32),
        ],
        scratch_shapes=[
            pltpu.VMEM((RB, NPAD), jnp.float32),
            pltpu.VMEM((RB, K), jnp.float32),
            pltpu.VMEM((RB, K), jnp.int32),
        ],
    )(boxes_p, boxesT)


def _prep_body(featsT, boxesT, w1ct, w1nt, w1dst, b1c, t_out, base_out):
    f = featsT[...]
    bx = boxesT[...][0:4, :]
    p = jnp.dot(w1dst[...], bx, preferred_element_type=jnp.float32)
    t_out[...] = jnp.dot(w1nt[...], f, preferred_element_type=jnp.float32) + p
    base_out[...] = (jnp.dot(w1ct[...], f, preferred_element_type=jnp.float32)
                     + b1c[...] - p)


def _prep_pallas(featsT, boxesT, w1ct, w1nt, w1dst, b1c):
    return pl.pallas_call(
        _prep_body,
        grid=(NPAD // CB,),
        in_specs=[
            pl.BlockSpec((D, CB), lambda i: (0, i)),
            pl.BlockSpec((8, CB), lambda i: (0, i)),
            pl.BlockSpec((H, D), lambda i: (0, 0)),
            pl.BlockSpec((H, D), lambda i: (0, 0)),
            pl.BlockSpec((H, 4), lambda i: (0, 0)),
            pl.BlockSpec((H, 1), lambda i: (0, 0)),
        ],
        out_specs=[
            pl.BlockSpec((H, CB), lambda i: (0, i)),
            pl.BlockSpec((H, CB), lambda i: (0, i)),
        ],
        out_shape=[
            jax.ShapeDtypeStruct((H, NPAD), jnp.float32),
            jax.ShapeDtypeStruct((H, NPAD), jnp.float32),
        ],
    )(featsT, boxesT, w1ct, w1nt, w1dst, b1c)


def _make_sc_gather():
    mesh = plsc.VectorSubcoreMesh(core_axis_name="c", subcore_axis_name="s")

    @functools.partial(
        pl.kernel,
        mesh=mesh,
        out_type=jax.ShapeDtypeStruct((K, H, NPAD), jnp.float32),
        compiler_params=pltpu.CompilerParams(needs_layout_passes=False),
        scratch_types=[
            pltpu.VMEM((FG, NPAD), jnp.float32),
            pltpu.VMEM((GCH,), jnp.int32),
            pltpu.VMEM((GCH,), jnp.int32),
            pltpu.VMEM((FG, GCH), jnp.float32),
            pltpu.VMEM((FG, GCH), jnp.float32),
            pltpu.VMEM((NCHP,), jnp.int32),
            pltpu.SemaphoreType.DMA,
            pltpu.SemaphoreType.DMA,
            pltpu.SemaphoreType.DMA,
            pltpu.SemaphoreType.DMA,
        ],
    )
    def sc_gather(tt_hbm, idx_hbm, flags_hbm, out_hbm, tslice, idx0, idx1,
                  obuf0, obuf1, flags_v, is0, is1, os0, os1):
        tid = lax.axis_index("s") * NC + lax.axis_index("c")
        g = tid // NHALF
        h = tid - g * NHALF
        pltpu.sync_copy(flags_hbm.at[h], flags_v)
        pltpu.sync_copy(tt_hbm.at[pl.ds(g * FG, FG)], tslice)

        def flag_at(ch):
            return flags_v[pl.ds(ch, 16)][0] > 0

        idxs = (idx0, idx1)
        obufs = (obuf0, obuf1)
        isems = (is0, is1)
        osems = (os0, os1)

        def gather_chunk(idx_v, obuf):
            for j in range(GCH // 16):
                idxv = idx_v[pl.ds(j * 16, 16)]
                for c in range(FG):
                    vals16 = plsc.load_gather(
                        tslice, [jnp.full((16,), c, jnp.int32), idxv])
                    obuf[c, pl.ds(j * 16, 16)] = vals16

        def out_slice(ch):
            k = ch // SUBS
            sub = ch - k * SUBS
            return out_hbm.at[k, pl.ds(g * FG, FG),
                              pl.ds(h * HALF + sub * GCH, GCH)]

        @pl.when(flag_at(0))
        def _():
            pltpu.async_copy(idx_hbm.at[h, 0], idx0, is0)

        def pair(i, carry):
            fs = []
            for u in range(2):
                ch = 2 * i + u
                f_ch = flag_at(ch)
                fs.append((u, ch, f_ch))

                @pl.when(f_ch)
                def _():
                    pltpu.make_async_copy(idx_hbm.at[h, ch], idxs[u],
                                          isems[u]).wait()

                nxt = ch + 1

                @pl.when(jnp.logical_and(nxt < NCH, flag_at(nxt)))
                def _():
                    pltpu.async_copy(idx_hbm.at[h, nxt], idxs[1 - u],
                                     isems[1 - u])

                @pl.when(f_ch)
                def _():
                    gather_chunk(idxs[u], obufs[u])
                    pltpu.async_copy(obufs[u], out_slice(ch), osems[u])

            for u, ch, f_ch in fs:

                @pl.when(f_ch)
                def _():
                    pltpu.make_async_copy(obufs[u], out_slice(ch),
                                          osems[u]).wait()
            return carry

        lax.fori_loop(0, NCH // 2, pair, 0)

    return sc_gather


@functools.cache
def _get_sc_gather():
    return _make_sc_gather()


def _sc_gather(tt, idx_sc, flags):
    return _get_sc_gather()(tt, idx_sc, flags)


def _make_mlp(final):
    def body(nt3, valsT, baseT, featsT, w1vc, w2t, b2c, wot, boc, wfc, bfc,
             out_ref):
        basev = baseT[...]
        w1vv = w1vc[...]
        w2v = w2t[...]
        b2v = b2c[...]
        pooled = None
        for k in range(K):
            ntk = nt3[k]
            vk = valsT[k:k + 1, :]
            h1 = jnp.maximum(ntk + basev + w1vv * vk, 0.0)
            h2 = jnp.maximum(
                jnp.dot(w2v, h1, preferred_element_type=jnp.float32) + b2v,
                0.0)
            h2 = jnp.where(vk > THR, h2, -1e30)
            pooled = h2 if k == 0 else jnp.maximum(pooled, h2)
        pooled = jnp.maximum(pooled, 0.0)
        newf = (featsT[...]
                + jnp.dot(wot[...], pooled, preferred_element_type=jnp.float32)
                + boc[...])
        if final:
            s = jnp.sum(newf * wfc[...], axis=0, keepdims=True) + bfc[...]
            out_ref[...] = jnp.broadcast_to(s, (8, RB2))
        else:
            out_ref[...] = newf
    return body


def _mlp_pallas(nt3, valsT, baseT, featsT, w1vc, w2t, b2c, wot, boc, wfc, bfc,
                final):
    out_rows = 8 if final else D
    return pl.pallas_call(
        _make_mlp(final),
        grid=(NPAD // RB2,),
        in_specs=[
            pl.BlockSpec((K, H, RB2), lambda i: (0, 0, i)),
            pl.BlockSpec((K, RB2), lambda i: (0, i)),
            pl.BlockSpec((H, RB2), lambda i: (0, i)),
            pl.BlockSpec((D, RB2), lambda i: (0, i)),
            pl.BlockSpec((H, 1), lambda i: (0, 0)),
            pl.BlockSpec((H, H), lambda i: (0, 0)),
            pl.BlockSpec((H, 1), lambda i: (0, 0)),
            pl.BlockSpec((D, H), lambda i: (0, 0)),
            pl.BlockSpec((D, 1), lambda i: (0, 0)),
            pl.BlockSpec((D, 1), lambda i: (0, 0)),
            pl.BlockSpec((1, 1), lambda i: (0, 0)),
        ],
        out_specs=pl.BlockSpec((out_rows, RB2), lambda i: (0, i)),
        out_shape=jax.ShapeDtypeStruct((out_rows, NPAD), jnp.float32),
    )(nt3, valsT, baseT, featsT, w1vc, w2t, b2c, wot, boc, wfc, bfc)


def kernel(interpolated, rpn_boxes,
           W1_0, b1_0, W2_0, b2_0, Wo_0, bo_0,
           W1_1, b1_1, W2_1, b2_1, Wo_1, bo_1,
           Wf, bf):
    boxes_p = jnp.zeros((NPAD, 4), jnp.float32).at[:N].set(rpn_boxes)
    boxesT = jnp.zeros((8, NPAD), jnp.float32).at[:4].set(boxes_p.T)
    vals, idx = _topk_pallas(boxes_p, boxesT)
    valsT = vals.T
    idx_sc = (idx.T.reshape(K, NHALF, SUBS, GCH)
              .transpose(1, 0, 2, 3).reshape(NHALF, NCH, GCH))
    act = ((valsT > THR).reshape(K, NHALF, SUBS, GCH).any(axis=-1)
           .transpose(1, 0, 2).reshape(NHALF, NCH).astype(jnp.int32))
    flags = jnp.zeros((NHALF, NCHP), jnp.int32).at[:, :NCH].set(act)

    featsT = jnp.zeros((D, NPAD), jnp.float32).at[:, :N].set(interpolated.T)
    bfc = bf.reshape(1, 1)
    params = [(W1_0, b1_0, W2_0, b2_0, Wo_0, bo_0),
              (W1_1, b1_1, W2_1, b2_1, Wo_1, bo_1)]
    out = None
    for b, (W1, b1, W2, b2, Wo, bo) in enumerate(params):
        w1ct = W1[:D].T
        w1nt = W1[D:2 * D].T
        w1dst = (W1[2 * D:2 * D + 4] / TILE).T
        w1vc = W1[2 * D + 4].reshape(H, 1)
        tt, baseT = _prep_pallas(featsT, boxesT, w1ct, w1nt, w1dst,
                                 b1.reshape(H, 1))
        nt3 = _sc_gather(tt, idx_sc, flags)
        final = b == len(params) - 1
        res = _mlp_pallas(nt3, valsT, baseT, featsT, w1vc, W2.T,
                          b2.reshape(H, 1), Wo.T, bo.reshape(D, 1), Wf, bfc,
                          final)
        if final:
            out = res
        else:
            featsT = res
    return out[0:1, :N].T


# speedup vs baseline: 1.1409x; 1.0955x over previous
import functools

import jax
import jax.numpy as jnp
from jax import lax
from jax.experimental import pallas as pl
from jax.experimental.pallas import tpu as pltpu
from jax.experimental.pallas import tpu_sc as plsc

N = 5000
NPAD = 5120
RB = 256
RB2 = 128
CB = 640
D = 129
K = 32
H = 256
TILE = 224.0
THR = 0.5
BIGI = 2 ** 30

NC, NS = 2, 16
FG = 16
NHALF = 2
HALF = NPAD // NHALF
GCH = 128
SUBS = HALF // GCH
NCH = K * SUBS
NCHP = NCH + 16


def _topk_body(boxes_blk, boxesT, vals_out, idx_out, iou_s, vacc_s, iacc_s):
    rows = boxes_blk[...]
    bT = boxesT[...]
    x1c, y1c, x2c, y2c = bT[0:1, :], bT[1:2, :], bT[2:3, :], bT[3:4, :]
    x1r, y1r = rows[:, 0:1], rows[:, 1:2]
    x2r, y2r = rows[:, 2:3], rows[:, 3:4]
    inter = (jnp.maximum(jnp.minimum(x2r, x2c) - jnp.maximum(x1r, x1c), 0.0)
             * jnp.maximum(jnp.minimum(y2r, y2c) - jnp.maximum(y1r, y1c), 0.0))
    area_r = (x2r - x1r) * (y2r - y1r)
    area_c = (x2c - x1c) * (y2c - y1c)
    iou = inter / (area_r + area_c - inter + 1e-9)
    iou_s[...] = jnp.where(iou > THR, iou, -1.0)
    vacc_s[...] = jnp.full((RB, K), -1.0, dtype=jnp.float32)
    iacc_s[...] = jnp.zeros((RB, K), dtype=jnp.int32)

    colids = jax.lax.broadcasted_iota(jnp.int32, (RB, NPAD), 1)
    kio = jax.lax.broadcasted_iota(jnp.int32, (RB, K), 1)

    def body(carry):
        k, _ = carry
        cur = iou_s[...]
        m = jnp.max(cur, axis=1, keepdims=True)
        t = jnp.where(cur == m, colids, BIGI)
        a = jnp.min(t, axis=1, keepdims=True)
        vacc_s[...] = jnp.where(kio == k, m, vacc_s[...])
        iacc_s[...] = jnp.where(kio == k, a, iacc_s[...])
        iou_s[...] = jnp.where(colids == a, -1.0, cur)
        return k + 1, jnp.max(m) > 0.0

    def cond(carry):
        k, active = carry
        return jnp.logical_and(k < K, active)

    lax.while_loop(cond, body, (jnp.int32(0), True))
    vals_out[...] = vacc_s[...]
    idx_out[...] = iacc_s[...]


def _topk_pallas(boxes_p, boxesT):
    return pl.pallas_call(
        _topk_body,
        grid=(NPAD // RB,),
        in_specs=[
            pl.BlockSpec((RB, 4), lambda i: (i, 0)),
            pl.BlockSpec((8, NPAD), lambda i: (0, 0)),
        ],
        out_specs=[
            pl.BlockSpec((RB, K), lambda i: (i, 0)),
            pl.BlockSpec((RB, K), lambda i: (i, 0)),
        ],
        out_shape=[
            jax.ShapeDtypeStruct((NPAD, K), jnp.float32),
            jax.ShapeDtypeStruct((NPAD, K), jnp.int32),
        ],
        scratch_shapes=[
            pltpu.VMEM((RB, NPAD), jnp.float32),
            pltpu.VMEM((RB, K), jnp.float32),
            pltpu.VMEM((RB, K), jnp.int32),
        ],
    )(boxes_p, boxesT)


def _prep_body(featsT, boxesT, w1ct, w1nt, w1dst, b1c, t_out, base_out):
    f = featsT[...]
    bx = boxesT[...][0:4, :]
    p = jnp.dot(w1dst[...], bx, preferred_element_type=jnp.float32)
    t_out[...] = jnp.dot(w1nt[...], f, preferred_element_type=jnp.float32) + p
    base_out[...] = (jnp.dot(w1ct[...], f, preferred_element_type=jnp.float32)
                     + b1c[...] - p)


def _prep_pallas(featsT, boxesT, w1ct, w1nt, w1dst, b1c):
    return pl.pallas_call(
        _prep_body,
        grid=(NPAD // CB,),
        in_specs=[
            pl.BlockSpec((D, CB), lambda i: (0, i)),
            pl.BlockSpec((8, CB), lambda i: (0, i)),
            pl.BlockSpec((H, D), lambda i: (0, 0)),
            pl.BlockSpec((H, D), lambda i: (0, 0)),
            pl.BlockSpec((H, 4), lambda i: (0, 0)),
            pl.BlockSpec((H, 1), lambda i: (0, 0)),
        ],
        out_specs=[
            pl.BlockSpec((H, CB), lambda i: (0, i)),
            pl.BlockSpec((H, CB), lambda i: (0, i)),
        ],
        out_shape=[
            jax.ShapeDtypeStruct((H, NPAD), jnp.float32),
            jax.ShapeDtypeStruct((H, NPAD), jnp.float32),
        ],
    )(featsT, boxesT, w1ct, w1nt, w1dst, b1c)


def _make_sc_gather():
    mesh = plsc.VectorSubcoreMesh(core_axis_name="c", subcore_axis_name="s")

    @functools.partial(
        pl.kernel,
        mesh=mesh,
        out_type=jax.ShapeDtypeStruct((K, H, NPAD), jnp.float32),
        compiler_params=pltpu.CompilerParams(needs_layout_passes=False),
        scratch_types=[
            pltpu.VMEM((FG, NPAD), jnp.float32),
            pltpu.VMEM((GCH,), jnp.int32),
            pltpu.VMEM((GCH,), jnp.int32),
            pltpu.VMEM((FG, GCH), jnp.float32),
            pltpu.VMEM((FG, GCH), jnp.float32),
            pltpu.VMEM((NCHP,), jnp.int32),
            pltpu.SemaphoreType.DMA,
            pltpu.SemaphoreType.DMA,
            pltpu.SemaphoreType.DMA,
            pltpu.SemaphoreType.DMA,
        ],
    )
    def sc_gather(tt_hbm, idx_hbm, flags_hbm, out_hbm, tslice, idx0, idx1,
                  obuf0, obuf1, flags_v, is0, is1, os0, os1):
        tid = lax.axis_index("s") * NC + lax.axis_index("c")
        g = tid // NHALF
        h = tid - g * NHALF
        pltpu.sync_copy(flags_hbm.at[h], flags_v)
        pltpu.sync_copy(tt_hbm.at[pl.ds(g * FG, FG)], tslice)

        def flag_at(ch):
            return flags_v[pl.ds(ch, 16)][0] > 0

        idxs = (idx0, idx1)
        obufs = (obuf0, obuf1)
        isems = (is0, is1)
        osems = (os0, os1)

        def gather_chunk(idx_v, obuf):
            for j in range(GCH // 16):
                idxv = idx_v[pl.ds(j * 16, 16)]
                for c in range(FG):
                    vals16 = plsc.load_gather(
                        tslice, [jnp.full((16,), c, jnp.int32), idxv])
                    obuf[c, pl.ds(j * 16, 16)] = vals16

        def out_slice(ch):
            k = ch // SUBS
            sub = ch - k * SUBS
            return out_hbm.at[k, pl.ds(g * FG, FG),
                              pl.ds(h * HALF + sub * GCH, GCH)]

        @pl.when(flag_at(0))
        def _():
            pltpu.async_copy(idx_hbm.at[h, 0], idx0, is0)

        def pair(i, carry):
            fs = []
            for u in range(2):
                ch = 2 * i + u
                f_ch = flag_at(ch)
                fs.append((u, ch, f_ch))

                @pl.when(f_ch)
                def _():
                    pltpu.make_async_copy(idx_hbm.at[h, ch], idxs[u],
                                          isems[u]).wait()

                nxt = ch + 1

                @pl.when(jnp.logical_and(nxt < NCH, flag_at(nxt)))
                def _():
                    pltpu.async_copy(idx_hbm.at[h, nxt], idxs[1 - u],
                                     isems[1 - u])

                @pl.when(f_ch)
                def _():
                    gather_chunk(idxs[u], obufs[u])
                    pltpu.async_copy(obufs[u], out_slice(ch), osems[u])

            for u, ch, f_ch in fs:

                @pl.when(f_ch)
                def _():
                    pltpu.make_async_copy(obufs[u], out_slice(ch),
                                          osems[u]).wait()
            return carry

        lax.fori_loop(0, NCH // 2, pair, 0)

    return sc_gather


@functools.cache
def _get_sc_gather():
    return _make_sc_gather()


def _sc_gather(tt, idx_sc, flags):
    return _get_sc_gather()(tt, idx_sc, flags)


def _make_mlp(final):
    def body(nt3, valsT, baseT, featsT, w1vc, w2t, b2c, wot, boc, wfc, bfc,
             out_ref):
        basev = baseT[...]
        w1vv = w1vc[...]
        w2v = w2t[...]
        b2v = b2c[...]
        pooled = None
        for k in range(K):
            ntk = nt3[k]
            vk = valsT[k:k + 1, :]
            h1 = jnp.maximum(ntk + basev + w1vv * vk, 0.0)
            h2 = jnp.maximum(
                jnp.dot(w2v, h1, preferred_element_type=jnp.float32) + b2v,
                0.0)
            h2 = jnp.where(vk > THR, h2, -1e30)
            pooled = h2 if k == 0 else jnp.maximum(pooled, h2)
        pooled = jnp.maximum(pooled, 0.0)
        newf = (featsT[...]
                + jnp.dot(wot[...], pooled, preferred_element_type=jnp.float32)
                + boc[...])
        if final:
            s = jnp.sum(newf * wfc[...], axis=0, keepdims=True) + bfc[...]
            out_ref[...] = jnp.broadcast_to(s, (8, RB2))
        else:
            out_ref[...] = newf
    return body


def _mlp_pallas(nt3, valsT, baseT, featsT, w1vc, w2t, b2c, wot, boc, wfc, bfc,
                final):
    out_rows = 8 if final else D
    return pl.pallas_call(
        _make_mlp(final),
        grid=(NPAD // RB2,),
        in_specs=[
            pl.BlockSpec((K, H, RB2), lambda i: (0, 0, i)),
            pl.BlockSpec((K, RB2), lambda i: (0, i)),
            pl.BlockSpec((H, RB2), lambda i: (0, i)),
            pl.BlockSpec((D, RB2), lambda i: (0, i)),
            pl.BlockSpec((H, 1), lambda i: (0, 0)),
            pl.BlockSpec((H, H), lambda i: (0, 0)),
            pl.BlockSpec((H, 1), lambda i: (0, 0)),
            pl.BlockSpec((D, H), lambda i: (0, 0)),
            pl.BlockSpec((D, 1), lambda i: (0, 0)),
            pl.BlockSpec((D, 1), lambda i: (0, 0)),
            pl.BlockSpec((1, 1), lambda i: (0, 0)),
        ],
        out_specs=pl.BlockSpec((out_rows, RB2), lambda i: (0, i)),
        out_shape=jax.ShapeDtypeStruct((out_rows, NPAD), jnp.float32),
    )(nt3, valsT, baseT, featsT, w1vc, w2t, b2c, wot, boc, wfc, bfc)


def kernel(interpolated, rpn_boxes,
           W1_0, b1_0, W2_0, b2_0, Wo_0, bo_0,
           W1_1, b1_1, W2_1, b2_1, Wo_1, bo_1,
           Wf, bf):
    boxes_p = jnp.zeros((NPAD, 4), jnp.float32).at[:N].set(rpn_boxes)
    boxesT = jnp.zeros((8, NPAD), jnp.float32).at[:4].set(boxes_p.T)
    vals, idx = _topk_pallas(boxes_p, boxesT)
    valsT = vals.T
    idx_sc = (idx.T.reshape(K, NHALF, SUBS, GCH)
              .transpose(1, 0, 2, 3).reshape(NHALF, NCH, GCH))
    act = ((valsT > THR).reshape(K, NHALF, SUBS, GCH).any(axis=-1)
           .transpose(1, 0, 2).reshape(NHALF, NCH).astype(jnp.int32))
    flags = jnp.zeros((NHALF, NCHP), jnp.int32).at[:, :NCH].set(act)

    featsT = jnp.zeros((D, NPAD), jnp.float32).at[:, :N].set(interpolated.T)
    bfc = bf.reshape(1, 1)
    params = [(W1_0, b1_0, W2_0, b2_0, Wo_0, bo_0),
              (W1_1, b1_1, W2_1, b2_1, Wo_1, bo_1)]
    out = None
    for b, (W1, b1, W2, b2, Wo, bo) in enumerate(params):
        w1ct = W1[:D].T
        w1nt = W1[D:2 * D].T
        w1dst = (W1[2 * D:2 * D + 4] / TILE).T
        w1vc = W1[2 * D + 4].reshape(H, 1)
        tt, baseT = _prep_pallas(featsT, boxesT, w1ct, w1nt, w1dst,
                                 b1.reshape(H, 1))
        nt3 = _sc_gather(tt, idx_sc, flags)
        final = b == len(params) - 1
        res = _mlp_pallas(nt3, valsT, baseT, featsT, w1vc, W2.T,
                          b2.reshape(H, 1), Wo.T, bo.reshape(D, 1), Wf, bfc,
                          final)
        if final:
            out = res
        else:
            featsT = res
    return out[0:1, :N].T
